# baseline (speedup 1.0000x reference)
import jax
import jax.numpy as jnp
from jax.experimental import pallas as pl
from jax.experimental.pallas import tpu as pltpu

_ROWS = 1024
_HALF = 512
_NBUF = 3


def _gate_kernel(x_auto_ref, x_hbm, wt_ref, out_ref, x_buf, sems):
    i = pl.program_id(0)
    n = pl.num_programs(0)

    def copy(step, slot):
        return pltpu.make_async_copy(
            x_hbm.at[pl.ds(step * _ROWS + _HALF, _HALF), :],
            x_buf.at[slot],
            sems.at[slot],
        )

    @pl.when(i == 0)
    def _():
        for b in range(_NBUF):
            copy(b, b).start()

    def softmax_rows(x):
        logits = jnp.dot(x, wt_ref[...],
                         preferred_element_type=jnp.float32,
                         precision=jax.lax.Precision.DEFAULT)
        m = jnp.max(logits, axis=1, keepdims=True)
        e = jnp.exp(logits - m)
        return e / jnp.sum(e, axis=1, keepdims=True)

    out_ref[pl.ds(0, _HALF), :] = softmax_rows(x_auto_ref[...])

    slot = jax.lax.rem(i, _NBUF)
    copy(i, slot).wait()
    out_ref[pl.ds(_HALF, _HALF), :] = softmax_rows(x_buf[slot])

    @pl.when(i + _NBUF < n)
    def _():
        copy(i + _NBUF, slot).start()


def kernel(inputs, W):
    tokens, d = inputs.shape
    n_exp = W.shape[0]
    wt = W.T
    return pl.pallas_call(
        _gate_kernel,
        grid=(tokens // _ROWS,),
        in_specs=[
            pl.BlockSpec((_HALF, d), lambda i: (2 * i, 0)),
            pl.BlockSpec(memory_space=pl.ANY),
            pl.BlockSpec((d, n_exp), lambda i: (0, 0)),
        ],
        out_specs=pl.BlockSpec((_ROWS, n_exp), lambda i: (i, 0)),
        out_shape=jax.ShapeDtypeStruct((tokens, n_exp), jnp.float32),
        scratch_shapes=[
            pltpu.VMEM((_NBUF, _HALF, d), jnp.float32),
            pltpu.SemaphoreType.DMA((_NBUF,)),
        ],
        compiler_params=pltpu.CompilerParams(
            dimension_semantics=("arbitrary",),
        ),
    )(inputs, inputs, wt)


# speedup vs baseline: 1.0993x; 1.0993x over previous
import jax
import jax.numpy as jnp
from jax.experimental import pallas as pl
from jax.experimental.pallas import tpu as pltpu

_BM = 1024


def _gate_kernel(x_ref, wt_ref, out_ref):
    logits = jnp.dot(x_ref[...].astype(jnp.bfloat16), wt_ref[...],
                     preferred_element_type=jnp.float32)
    m = jnp.max(logits, axis=1, keepdims=True)
    e = jnp.exp(logits - m)
    out_ref[...] = e / jnp.sum(e, axis=1, keepdims=True)


def kernel(inputs, W):
    tokens, d = inputs.shape
    n_exp = W.shape[0]
    wt = W.T.astype(jnp.bfloat16)
    return pl.pallas_call(
        _gate_kernel,
        grid=(tokens // _BM,),
        in_specs=[
            pl.BlockSpec((_BM, d), lambda i: (i, 0)),
            pl.BlockSpec((d, n_exp), lambda i: (0, 0)),
        ],
        out_specs=pl.BlockSpec((_BM, n_exp), lambda i: (i, 0)),
        out_shape=jax.ShapeDtypeStruct((tokens, n_exp), jnp.float32),
        compiler_params=pltpu.CompilerParams(
            dimension_semantics=("arbitrary",),
        ),
    )(inputs, wt)


# speedup vs baseline: 1.1034x; 1.0037x over previous
import jax
import jax.numpy as jnp
from jax.experimental import pallas as pl
from jax.experimental.pallas import tpu as pltpu

_BM = 1024


def _gate_kernel(x_ref, w_ref, out_ref):
    logits = jax.lax.dot_general(
        x_ref[...].astype(jnp.bfloat16), w_ref[...],
        dimension_numbers=(((1,), (1,)), ((), ())),
        preferred_element_type=jnp.float32)
    m = jnp.max(logits, axis=1, keepdims=True)
    e = jnp.exp(logits - m)
    out_ref[...] = e / jnp.sum(e, axis=1, keepdims=True)


def kernel(inputs, W):
    tokens, d = inputs.shape
    n_exp = W.shape[0]
    wb = W.astype(jnp.bfloat16)
    return pl.pallas_call(
        _gate_kernel,
        grid=(tokens // _BM,),
        in_specs=[
            pl.BlockSpec((_BM, d), lambda i: (i, 0)),
            pl.BlockSpec((n_exp, d), lambda i: (0, 0)),
        ],
        out_specs=pl.BlockSpec((_BM, n_exp), lambda i: (i, 0)),
        out_shape=jax.ShapeDtypeStruct((tokens, n_exp), jnp.float32),
        compiler_params=pltpu.CompilerParams(
            dimension_semantics=("arbitrary",),
        ),
    )(inputs, wb)
